# baseline (speedup 1.0000x reference)
import functools

import jax
import jax.numpy as jnp
from jax import lax
from jax.experimental import pallas as pl
from jax.experimental.pallas import tpu as pltpu

_EB = 512
_INTERPRET = False


def _mm_first_body(x_ref, w_ref, as_ref, ad_ref, h_ref, aso_ref, ado_ref):
    h = jnp.dot(x_ref[...], w_ref[...], preferred_element_type=jnp.float32)
    h_ref[...] = h
    aso_ref[...] = jnp.dot(h, as_ref[...], preferred_element_type=jnp.float32)
    ado_ref[...] = jnp.dot(h, ad_ref[...], preferred_element_type=jnp.float32)


def _mm_elu_body(x_ref, b_ref, w_ref, as_ref, ad_ref, h_ref, aso_ref, ado_ref):
    v = x_ref[...] + b_ref[...]
    xin = jnp.where(v > 0, v, jnp.exp(v) - 1.0)
    h = jnp.dot(xin, w_ref[...], preferred_element_type=jnp.float32)
    h_ref[...] = h
    aso_ref[...] = jnp.dot(h, as_ref[...], preferred_element_type=jnp.float32)
    ado_ref[...] = jnp.dot(h, ad_ref[...], preferred_element_type=jnp.float32)


def _mm_call(xin, W, As64, Ad64, bias, NB):
    n, fin = xin.shape
    d = W.shape[1]
    grid = n // NB
    row = pl.BlockSpec((NB, fin), lambda i: (i, 0))
    full = lambda s: pl.BlockSpec(s, lambda i: (0, 0))
    out = pl.BlockSpec((NB, d), lambda i: (i, 0))
    shapes = [jax.ShapeDtypeStruct((n, d), jnp.float32)] * 3
    if bias is None:
        return pl.pallas_call(
            _mm_first_body,
            grid=(grid,),
            in_specs=[row, full((fin, d)), full((d, d)), full((d, d))],
            out_specs=[out, out, out],
            out_shape=shapes,
            interpret=_INTERPRET,
        )(xin, W, As64, Ad64)
    return pl.pallas_call(
        _mm_elu_body,
        grid=(grid,),
        in_specs=[row, full((1, d)), full((fin, d)), full((d, d)), full((d, d))],
        out_specs=[out, out, out],
        out_shape=shapes,
        interpret=_INTERPRET,
    )(xin, bias, W, As64, Ad64)


def _denom_body(src_ref, dst_ref, cat_ref, den_ref, ex_ref):
    @pl.when(pl.program_id(0) == 0)
    def _():
        den_ref[...] = jnp.zeros_like(den_ref)

    def body(i, carry):
        s = src_ref[0, 0, i]
        t = dst_ref[0, 0, i]
        vs = cat_ref[pl.ds(s, 1), :]
        vt = cat_ref[pl.ds(t, 1), :]
        a = vs[:, :64] + vt[:, 64:]
        a = jnp.where(a > 0, a, 0.2 * a)
        ex = jnp.exp(a)
        ex_ref[pl.ds(i, 1), :] = ex
        den_ref[pl.ds(t, 1), :] = den_ref[pl.ds(t, 1), :] + ex
        return carry

    lax.fori_loop(0, _EB, body, 0)


def _agg_body(src_ref, dst_ref, hd_ref, ex_ref, out_ref):
    @pl.when(pl.program_id(0) == 0)
    def _():
        out_ref[...] = jnp.zeros_like(out_ref)

    def body(i, carry):
        s = src_ref[0, 0, i]
        t = dst_ref[0, 0, i]
        vs = hd_ref[pl.ds(s, 1), :]
        vt = hd_ref[pl.ds(t, 1), :]
        attn = ex_ref[pl.ds(i, 1), :] / (vt[:, 64:] + 1e-16)
        out_ref[pl.ds(t, 1), :] = (
            out_ref[pl.ds(t, 1), :] + vs[:, :64] * attn
        )
        return carry

    lax.fori_loop(0, _EB, body, 0)


def _denom_call(src3, dst3, cat):
    np_ = cat.shape[0]
    n_eb = src3.shape[0]
    idx = pl.BlockSpec((1, 1, _EB), lambda i: (i, 0, 0), memory_space=pltpu.SMEM)
    return pl.pallas_call(
        _denom_body,
        grid=(n_eb,),
        in_specs=[idx, idx, pl.BlockSpec((np_, 128), lambda i: (0, 0))],
        out_specs=[
            pl.BlockSpec((np_, 64), lambda i: (0, 0)),
            pl.BlockSpec((_EB, 64), lambda i: (i, 0)),
        ],
        out_shape=[
            jax.ShapeDtypeStruct((np_, 64), jnp.float32),
            jax.ShapeDtypeStruct((n_eb * _EB, 64), jnp.float32),
        ],
        interpret=_INTERPRET,
    )(src3, dst3, cat)


def _agg_call(src3, dst3, hd, ex):
    np_ = hd.shape[0]
    n_eb = src3.shape[0]
    idx = pl.BlockSpec((1, 1, _EB), lambda i: (i, 0, 0), memory_space=pltpu.SMEM)
    return pl.pallas_call(
        _agg_body,
        grid=(n_eb,),
        in_specs=[
            idx, idx,
            pl.BlockSpec((np_, 128), lambda i: (0, 0)),
            pl.BlockSpec((_EB, 64), lambda i: (i, 0)),
        ],
        out_specs=pl.BlockSpec((np_, 64), lambda i: (0, 0)),
        out_shape=jax.ShapeDtypeStruct((np_, 64), jnp.float32),
        interpret=_INTERPRET,
    )(src3, dst3, hd, ex)


def _make_pool_body(ngrid, nb, ngraphs):
    def body(agg_ref, b_ref, batch_ref, linw_ref, linb_ref, out_ref,
             ps_ref, cnt_ref):
        @pl.when(pl.program_id(0) == 0)
        def _():
            ps_ref[...] = jnp.zeros_like(ps_ref)
            cnt_ref[...] = jnp.zeros_like(cnt_ref)

        v = agg_ref[...] + b_ref[...]
        hb = jnp.where(v > 0, v, jnp.exp(v) - 1.0)
        ids = lax.broadcasted_iota(jnp.int32, (nb, ngraphs), 1)
        onehot = (batch_ref[...] == ids).astype(jnp.float32)
        dn = (((0,), (0,)), ((), ()))
        ps_ref[...] = ps_ref[...] + lax.dot_general(
            onehot, hb, dn, preferred_element_type=jnp.float32)
        cnt_ref[...] = cnt_ref[...] + lax.dot_general(
            onehot, jnp.ones((nb, 1), jnp.float32), dn,
            preferred_element_type=jnp.float32)

        @pl.when(pl.program_id(0) == ngrid - 1)
        def _():
            pooled = ps_ref[...] / jnp.maximum(cnt_ref[...], 1.0)
            logits = jnp.dot(pooled, linw_ref[...],
                             preferred_element_type=jnp.float32) + linb_ref[...]
            m = jnp.max(logits, axis=1, keepdims=True)
            lse = m + jnp.log(jnp.sum(jnp.exp(logits - m), axis=1,
                                      keepdims=True))
            out_ref[...] = logits - lse

    return body


def _pool_call(agg, bias, batchT, lin_W, lin_b, NB, ngraphs):
    n, d = agg.shape
    ngrid = n // NB
    ncls = lin_W.shape[1]
    full = lambda s: pl.BlockSpec(s, lambda i: (0, 0))
    return pl.pallas_call(
        _make_pool_body(ngrid, NB, ngraphs),
        grid=(ngrid,),
        in_specs=[
            pl.BlockSpec((NB, d), lambda i: (i, 0)),
            full((1, d)),
            pl.BlockSpec((NB, 1), lambda i: (i, 0)),
            full((d, ncls)),
            full((1, ncls)),
        ],
        out_specs=full((ngraphs, ncls)),
        out_shape=jax.ShapeDtypeStruct((ngraphs, ncls), jnp.float32),
        scratch_shapes=[
            pltpu.VMEM((ngraphs, d), jnp.float32),
            pltpu.VMEM((ngraphs, 1), jnp.float32),
        ],
        interpret=_INTERPRET,
    )(agg, bias, batchT, lin_W, lin_b)


def _alpha_mats(a_src, a_dst):
    h, c = a_src.shape
    d = h * c
    ii = jnp.arange(d)
    m = (ii[:, None] // c == ii[None, :] // c).astype(jnp.float32)
    return m * a_src.reshape(d)[:, None], m * a_dst.reshape(d)[:, None]


def kernel(x, edge_index, batch, W1, a_src1, a_dst1, b1, W2, a_src2, a_dst2,
           b2, W3, a_src3, a_dst3, b3, lin_W, lin_b):
    n = x.shape[0]
    d = W1.shape[1]
    e = edge_index.shape[1]
    ngraphs = 128
    NB = 1000 if n % 1000 == 0 else n
    NP = n + 8

    loop = jnp.arange(n, dtype=edge_index.dtype)
    src = jnp.concatenate([edge_index[0], loop])
    dst = jnp.concatenate([edge_index[1], loop])
    etot = e + n
    n_eb = -(-etot // _EB)
    padn = n_eb * _EB - etot
    src3 = jnp.concatenate(
        [src, jnp.zeros((padn,), src.dtype)]).reshape(n_eb, 1, _EB)
    dst3 = jnp.concatenate(
        [dst, jnp.full((padn,), n, dst.dtype)]).reshape(n_eb, 1, _EB)

    pad = lambda a: jnp.pad(a, ((0, NP - n), (0, 0)))

    layers = [
        (W1, a_src1, a_dst1, None),
        (W2, a_src2, a_dst2, b1.reshape(1, d)),
        (W3, a_src3, a_dst3, b2.reshape(1, d)),
    ]
    hin = x
    agg = None
    for W, a_s, a_d, bprev in layers:
        As64, Ad64 = _alpha_mats(a_s, a_d)
        h, als, ald = _mm_call(hin, W, As64, Ad64, bprev, NB)
        cat = jnp.concatenate([pad(als), pad(ald)], axis=1)
        den, ex = _denom_call(src3, dst3, cat)
        hd = jnp.concatenate([pad(h), den], axis=1)
        agg = _agg_call(src3, dst3, hd, ex)
        hin = agg[:n]

    return _pool_call(agg[:n], b3.reshape(1, d), batch.reshape(n, 1),
                      lin_W, lin_b.reshape(1, -1), NB, ngraphs)


# speedup vs baseline: 6.3397x; 6.3397x over previous
import functools

import jax
import jax.numpy as jnp
from jax import lax
from jax.experimental import pallas as pl
from jax.experimental.pallas import tpu as pltpu

_EB = 512
_INTERPRET = False


def _mm_first_body(x_ref, w_ref, as_ref, ad_ref, h_ref, aso_ref, ado_ref):
    h = jnp.dot(x_ref[...], w_ref[...], preferred_element_type=jnp.float32)
    h_ref[...] = h
    aso_ref[...] = jnp.dot(h, as_ref[...], preferred_element_type=jnp.float32)
    ado_ref[...] = jnp.dot(h, ad_ref[...], preferred_element_type=jnp.float32)


def _mm_elu_body(x_ref, b_ref, w_ref, as_ref, ad_ref, h_ref, aso_ref, ado_ref):
    v = x_ref[...] + b_ref[...]
    xin = jnp.where(v > 0, v, jnp.exp(v) - 1.0)
    h = jnp.dot(xin, w_ref[...], preferred_element_type=jnp.float32)
    h_ref[...] = h
    aso_ref[...] = jnp.dot(h, as_ref[...], preferred_element_type=jnp.float32)
    ado_ref[...] = jnp.dot(h, ad_ref[...], preferred_element_type=jnp.float32)


def _mm_call(xin, W, As64, Ad64, bias, NB):
    n, fin = xin.shape
    d = W.shape[1]
    grid = n // NB
    row = pl.BlockSpec((NB, fin), lambda i: (i, 0))
    full = lambda s: pl.BlockSpec(s, lambda i: (0, 0))
    out = pl.BlockSpec((NB, d), lambda i: (i, 0))
    shapes = [jax.ShapeDtypeStruct((n, d), jnp.float32)] * 3
    if bias is None:
        return pl.pallas_call(
            _mm_first_body,
            grid=(grid,),
            in_specs=[row, full((fin, d)), full((d, d)), full((d, d))],
            out_specs=[out, out, out],
            out_shape=shapes,
            interpret=_INTERPRET,
        )(xin, W, As64, Ad64)
    return pl.pallas_call(
        _mm_elu_body,
        grid=(grid,),
        in_specs=[row, full((1, d)), full((fin, d)), full((d, d)), full((d, d))],
        out_specs=[out, out, out],
        out_shape=shapes,
        interpret=_INTERPRET,
    )(xin, bias, W, As64, Ad64)


def _denom_body(src_ref, dst_ref, cat_ref, den_ref, ex_ref):
    @pl.when(pl.program_id(0) == 0)
    def _():
        den_ref[...] = jnp.zeros_like(den_ref)

    def body(i, carry):
        s = src_ref[0, 0, i]
        t = dst_ref[0, 0, i]
        vs = cat_ref[pl.ds(s, 1), :]
        vt = cat_ref[pl.ds(t, 1), :]
        a = vs[:, :64] + vt[:, 64:]
        a = jnp.where(a > 0, a, 0.2 * a)
        ex = jnp.exp(a)
        ex_ref[pl.ds(i, 1), :] = ex
        den_ref[pl.ds(t, 1), :] = den_ref[pl.ds(t, 1), :] + ex
        return carry

    lax.fori_loop(0, _EB, body, 0, unroll=8)


def _agg_body(src_ref, dst_ref, hd_ref, ex_ref, out_ref):
    @pl.when(pl.program_id(0) == 0)
    def _():
        out_ref[...] = jnp.zeros_like(out_ref)

    def body(i, carry):
        s = src_ref[0, 0, i]
        t = dst_ref[0, 0, i]
        vs = hd_ref[pl.ds(s, 1), :]
        vt = hd_ref[pl.ds(t, 1), :]
        attn = ex_ref[pl.ds(i, 1), :] / (vt[:, 64:] + 1e-16)
        out_ref[pl.ds(t, 1), :] = (
            out_ref[pl.ds(t, 1), :] + vs[:, :64] * attn
        )
        return carry

    lax.fori_loop(0, _EB, body, 0, unroll=8)


def _denom_call(src3, dst3, cat):
    np_ = cat.shape[0]
    n_eb = src3.shape[0]
    idx = pl.BlockSpec((1, 1, _EB), lambda i: (i, 0, 0), memory_space=pltpu.SMEM)
    return pl.pallas_call(
        _denom_body,
        grid=(n_eb,),
        in_specs=[idx, idx, pl.BlockSpec((np_, 128), lambda i: (0, 0))],
        out_specs=[
            pl.BlockSpec((np_, 64), lambda i: (0, 0)),
            pl.BlockSpec((_EB, 64), lambda i: (i, 0)),
        ],
        out_shape=[
            jax.ShapeDtypeStruct((np_, 64), jnp.float32),
            jax.ShapeDtypeStruct((n_eb * _EB, 64), jnp.float32),
        ],
        interpret=_INTERPRET,
    )(src3, dst3, cat)


def _agg_call(src3, dst3, hd, ex):
    np_ = hd.shape[0]
    n_eb = src3.shape[0]
    idx = pl.BlockSpec((1, 1, _EB), lambda i: (i, 0, 0), memory_space=pltpu.SMEM)
    return pl.pallas_call(
        _agg_body,
        grid=(n_eb,),
        in_specs=[
            idx, idx,
            pl.BlockSpec((np_, 128), lambda i: (0, 0)),
            pl.BlockSpec((_EB, 64), lambda i: (i, 0)),
        ],
        out_specs=pl.BlockSpec((np_, 64), lambda i: (0, 0)),
        out_shape=jax.ShapeDtypeStruct((np_, 64), jnp.float32),
        interpret=_INTERPRET,
    )(src3, dst3, hd, ex)


def _make_pool_body(ngrid, nb, ngraphs):
    def body(agg_ref, b_ref, batch_ref, linw_ref, linb_ref, out_ref,
             ps_ref, cnt_ref):
        @pl.when(pl.program_id(0) == 0)
        def _():
            ps_ref[...] = jnp.zeros_like(ps_ref)
            cnt_ref[...] = jnp.zeros_like(cnt_ref)

        v = agg_ref[...] + b_ref[...]
        hb = jnp.where(v > 0, v, jnp.exp(v) - 1.0)
        ids = lax.broadcasted_iota(jnp.int32, (nb, ngraphs), 1)
        onehot = (batch_ref[...] == ids).astype(jnp.float32)
        dn = (((0,), (0,)), ((), ()))
        ps_ref[...] = ps_ref[...] + lax.dot_general(
            onehot, hb, dn, preferred_element_type=jnp.float32)
        cnt_ref[...] = cnt_ref[...] + lax.dot_general(
            onehot, jnp.ones((nb, 1), jnp.float32), dn,
            preferred_element_type=jnp.float32)

        @pl.when(pl.program_id(0) == ngrid - 1)
        def _():
            pooled = ps_ref[...] / jnp.maximum(cnt_ref[...], 1.0)
            logits = jnp.dot(pooled, linw_ref[...],
                             preferred_element_type=jnp.float32) + linb_ref[...]
            m = jnp.max(logits, axis=1, keepdims=True)
            lse = m + jnp.log(jnp.sum(jnp.exp(logits - m), axis=1,
                                      keepdims=True))
            out_ref[...] = logits - lse

    return body


def _pool_call(agg, bias, batchT, lin_W, lin_b, NB, ngraphs):
    n, d = agg.shape
    ngrid = n // NB
    ncls = lin_W.shape[1]
    full = lambda s: pl.BlockSpec(s, lambda i: (0, 0))
    return pl.pallas_call(
        _make_pool_body(ngrid, NB, ngraphs),
        grid=(ngrid,),
        in_specs=[
            pl.BlockSpec((NB, d), lambda i: (i, 0)),
            full((1, d)),
            pl.BlockSpec((NB, 1), lambda i: (i, 0)),
            full((d, ncls)),
            full((1, ncls)),
        ],
        out_specs=full((ngraphs, ncls)),
        out_shape=jax.ShapeDtypeStruct((ngraphs, ncls), jnp.float32),
        scratch_shapes=[
            pltpu.VMEM((ngraphs, d), jnp.float32),
            pltpu.VMEM((ngraphs, 1), jnp.float32),
        ],
        interpret=_INTERPRET,
    )(agg, bias, batchT, lin_W, lin_b)


def _alpha_mats(a_src, a_dst):
    h, c = a_src.shape
    d = h * c
    ii = jnp.arange(d)
    m = (ii[:, None] // c == ii[None, :] // c).astype(jnp.float32)
    return m * a_src.reshape(d)[:, None], m * a_dst.reshape(d)[:, None]


def kernel(x, edge_index, batch, W1, a_src1, a_dst1, b1, W2, a_src2, a_dst2,
           b2, W3, a_src3, a_dst3, b3, lin_W, lin_b):
    n = x.shape[0]
    d = W1.shape[1]
    e = edge_index.shape[1]
    ngraphs = 128
    NB = 1000 if n % 1000 == 0 else n
    NP = n + 8

    loop = jnp.arange(n, dtype=edge_index.dtype)
    src = jnp.concatenate([edge_index[0], loop])
    dst = jnp.concatenate([edge_index[1], loop])
    etot = e + n
    n_eb = -(-etot // _EB)
    padn = n_eb * _EB - etot
    src3 = jnp.concatenate(
        [src, jnp.zeros((padn,), src.dtype)]).reshape(n_eb, 1, _EB)
    dst3 = jnp.concatenate(
        [dst, jnp.full((padn,), n, dst.dtype)]).reshape(n_eb, 1, _EB)

    pad = lambda a: jnp.pad(a, ((0, NP - n), (0, 0)))

    layers = [
        (W1, a_src1, a_dst1, None),
        (W2, a_src2, a_dst2, b1.reshape(1, d)),
        (W3, a_src3, a_dst3, b2.reshape(1, d)),
    ]
    hin = x
    agg = None
    for W, a_s, a_d, bprev in layers:
        As64, Ad64 = _alpha_mats(a_s, a_d)
        h, als, ald = _mm_call(hin, W, As64, Ad64, bprev, NB)
        cat = jnp.concatenate([pad(als), pad(ald)], axis=1)
        den, ex = _denom_call(src3, dst3, cat)
        hd = jnp.concatenate([pad(h), den], axis=1)
        agg = _agg_call(src3, dst3, hd, ex)
        hin = agg[:n]

    return _pool_call(agg[:n], b3.reshape(1, d), batch.reshape(n, 1),
                      lin_W, lin_b.reshape(1, -1), NB, ngraphs)


# speedup vs baseline: 9.9034x; 1.5621x over previous
import functools

import jax
import jax.numpy as jnp
from jax import lax
from jax.experimental import pallas as pl
from jax.experimental.pallas import tpu as pltpu

_EB = 512
_INTERPRET = False


def _mm_first_body(x_ref, w_ref, as_ref, ad_ref, h_ref, aso_ref, ado_ref):
    h = jnp.dot(x_ref[...], w_ref[...], preferred_element_type=jnp.float32)
    h_ref[...] = h
    aso_ref[...] = jnp.dot(h, as_ref[...], preferred_element_type=jnp.float32)
    ado_ref[...] = jnp.dot(h, ad_ref[...], preferred_element_type=jnp.float32)


def _mm_elu_body(x_ref, b_ref, w_ref, as_ref, ad_ref, h_ref, aso_ref, ado_ref):
    v = x_ref[...] + b_ref[...]
    xin = jnp.where(v > 0, v, jnp.exp(v) - 1.0)
    h = jnp.dot(xin, w_ref[...], preferred_element_type=jnp.float32)
    h_ref[...] = h
    aso_ref[...] = jnp.dot(h, as_ref[...], preferred_element_type=jnp.float32)
    ado_ref[...] = jnp.dot(h, ad_ref[...], preferred_element_type=jnp.float32)


def _mm_call(xin, W, As64, Ad64, bias, NB):
    n, fin = xin.shape
    d = W.shape[1]
    grid = n // NB
    row = pl.BlockSpec((NB, fin), lambda i: (i, 0))
    full = lambda s: pl.BlockSpec(s, lambda i: (0, 0))
    out = pl.BlockSpec((NB, d), lambda i: (i, 0))
    shapes = [jax.ShapeDtypeStruct((n, d), jnp.float32)] * 3
    if bias is None:
        return pl.pallas_call(
            _mm_first_body,
            grid=(grid,),
            in_specs=[row, full((fin, d)), full((d, d)), full((d, d))],
            out_specs=[out, out, out],
            out_shape=shapes,
            interpret=_INTERPRET,
        )(xin, W, As64, Ad64)
    return pl.pallas_call(
        _mm_elu_body,
        grid=(grid,),
        in_specs=[row, full((1, d)), full((fin, d)), full((d, d)), full((d, d))],
        out_specs=[out, out, out],
        out_shape=shapes,
        interpret=_INTERPRET,
    )(xin, bias, W, As64, Ad64)


def _denom_body(src_ref, dst_ref, cat_ref, den_ref, ex_ref):
    @pl.when(pl.program_id(0) == 0)
    def _():
        den_ref[...] = jnp.zeros_like(den_ref)

    def body(i, carry):
        s = src_ref[0, 0, i]
        t = dst_ref[0, 0, i]
        vs = cat_ref[pl.ds(s, 1), :]
        vt = cat_ref[pl.ds(t, 1), :]
        a = vs[:, :64] + vt[:, 64:]
        a = jnp.where(a > 0, a, 0.2 * a)
        ex = jnp.exp(a)
        ex_ref[pl.ds(i, 1), :] = ex
        den_ref[pl.ds(t, 1), :] = den_ref[pl.ds(t, 1), :] + ex
        return carry

    lax.fori_loop(0, _EB, body, 0, unroll=16)


def _agg_body(src_ref, dst_ref, hd_ref, ex_ref, out_ref):
    @pl.when(pl.program_id(0) == 0)
    def _():
        out_ref[...] = jnp.zeros_like(out_ref)

    def body(i, carry):
        s = src_ref[0, 0, i]
        t = dst_ref[0, 0, i]
        vs = hd_ref[pl.ds(s, 1), :]
        vt = hd_ref[pl.ds(t, 1), :]
        attn = ex_ref[pl.ds(i, 1), :] / (vt[:, 64:] + 1e-16)
        out_ref[pl.ds(t, 1), :] = (
            out_ref[pl.ds(t, 1), :] + vs[:, :64] * attn
        )
        return carry

    lax.fori_loop(0, _EB, body, 0, unroll=16)


def _denom_call(src3, dst3, cat):
    np_ = cat.shape[0]
    n_eb = src3.shape[0]
    idx = pl.BlockSpec((1, 1, _EB), lambda i: (i, 0, 0), memory_space=pltpu.SMEM)
    return pl.pallas_call(
        _denom_body,
        grid=(n_eb,),
        in_specs=[idx, idx, pl.BlockSpec((np_, 128), lambda i: (0, 0))],
        out_specs=[
            pl.BlockSpec((np_, 64), lambda i: (0, 0)),
            pl.BlockSpec((_EB, 64), lambda i: (i, 0)),
        ],
        out_shape=[
            jax.ShapeDtypeStruct((np_, 64), jnp.float32),
            jax.ShapeDtypeStruct((n_eb * _EB, 64), jnp.float32),
        ],
        interpret=_INTERPRET,
    )(src3, dst3, cat)


def _agg_call(src3, dst3, hd, ex):
    np_ = hd.shape[0]
    n_eb = src3.shape[0]
    idx = pl.BlockSpec((1, 1, _EB), lambda i: (i, 0, 0), memory_space=pltpu.SMEM)
    return pl.pallas_call(
        _agg_body,
        grid=(n_eb,),
        in_specs=[
            idx, idx,
            pl.BlockSpec((np_, 128), lambda i: (0, 0)),
            pl.BlockSpec((_EB, 64), lambda i: (i, 0)),
        ],
        out_specs=pl.BlockSpec((np_, 64), lambda i: (0, 0)),
        out_shape=jax.ShapeDtypeStruct((np_, 64), jnp.float32),
        interpret=_INTERPRET,
    )(src3, dst3, hd, ex)


def _make_pool_body(ngrid, nb, ngraphs):
    def body(agg_ref, b_ref, batch_ref, linw_ref, linb_ref, out_ref,
             ps_ref, cnt_ref):
        @pl.when(pl.program_id(0) == 0)
        def _():
            ps_ref[...] = jnp.zeros_like(ps_ref)
            cnt_ref[...] = jnp.zeros_like(cnt_ref)

        v = agg_ref[...] + b_ref[...]
        hb = jnp.where(v > 0, v, jnp.exp(v) - 1.0)
        ids = lax.broadcasted_iota(jnp.int32, (nb, ngraphs), 1)
        onehot = (batch_ref[...] == ids).astype(jnp.float32)
        dn = (((0,), (0,)), ((), ()))
        ps_ref[...] = ps_ref[...] + lax.dot_general(
            onehot, hb, dn, preferred_element_type=jnp.float32)
        cnt_ref[...] = cnt_ref[...] + lax.dot_general(
            onehot, jnp.ones((nb, 1), jnp.float32), dn,
            preferred_element_type=jnp.float32)

        @pl.when(pl.program_id(0) == ngrid - 1)
        def _():
            pooled = ps_ref[...] / jnp.maximum(cnt_ref[...], 1.0)
            logits = jnp.dot(pooled, linw_ref[...],
                             preferred_element_type=jnp.float32) + linb_ref[...]
            m = jnp.max(logits, axis=1, keepdims=True)
            lse = m + jnp.log(jnp.sum(jnp.exp(logits - m), axis=1,
                                      keepdims=True))
            out_ref[...] = logits - lse

    return body


def _pool_call(agg, bias, batchT, lin_W, lin_b, NB, ngraphs):
    n, d = agg.shape
    ngrid = n // NB
    ncls = lin_W.shape[1]
    full = lambda s: pl.BlockSpec(s, lambda i: (0, 0))
    return pl.pallas_call(
        _make_pool_body(ngrid, NB, ngraphs),
        grid=(ngrid,),
        in_specs=[
            pl.BlockSpec((NB, d), lambda i: (i, 0)),
            full((1, d)),
            pl.BlockSpec((NB, 1), lambda i: (i, 0)),
            full((d, ncls)),
            full((1, ncls)),
        ],
        out_specs=full((ngraphs, ncls)),
        out_shape=jax.ShapeDtypeStruct((ngraphs, ncls), jnp.float32),
        scratch_shapes=[
            pltpu.VMEM((ngraphs, d), jnp.float32),
            pltpu.VMEM((ngraphs, 1), jnp.float32),
        ],
        interpret=_INTERPRET,
    )(agg, bias, batchT, lin_W, lin_b)


def _alpha_mats(a_src, a_dst):
    h, c = a_src.shape
    d = h * c
    ii = jnp.arange(d)
    m = (ii[:, None] // c == ii[None, :] // c).astype(jnp.float32)
    return m * a_src.reshape(d)[:, None], m * a_dst.reshape(d)[:, None]


def kernel(x, edge_index, batch, W1, a_src1, a_dst1, b1, W2, a_src2, a_dst2,
           b2, W3, a_src3, a_dst3, b3, lin_W, lin_b):
    n = x.shape[0]
    d = W1.shape[1]
    e = edge_index.shape[1]
    ngraphs = 128
    NB = 1000 if n % 1000 == 0 else n
    NP = n + 8

    loop = jnp.arange(n, dtype=edge_index.dtype)
    src = jnp.concatenate([edge_index[0], loop])
    dst = jnp.concatenate([edge_index[1], loop])
    etot = e + n
    n_eb = -(-etot // _EB)
    padn = n_eb * _EB - etot
    src3 = jnp.concatenate(
        [src, jnp.zeros((padn,), src.dtype)]).reshape(n_eb, 1, _EB)
    dst3 = jnp.concatenate(
        [dst, jnp.full((padn,), n, dst.dtype)]).reshape(n_eb, 1, _EB)

    pad = lambda a: jnp.pad(a, ((0, NP - n), (0, 0)))

    layers = [
        (W1, a_src1, a_dst1, None),
        (W2, a_src2, a_dst2, b1.reshape(1, d)),
        (W3, a_src3, a_dst3, b2.reshape(1, d)),
    ]
    hin = x
    agg = None
    for W, a_s, a_d, bprev in layers:
        As64, Ad64 = _alpha_mats(a_s, a_d)
        h, als, ald = _mm_call(hin, W, As64, Ad64, bprev, NB)
        cat = jnp.concatenate([pad(als), pad(ald)], axis=1)
        den, ex = _denom_call(src3, dst3, cat)
        hd = jnp.concatenate([pad(h), den], axis=1)
        agg = _agg_call(src3, dst3, hd, ex)
        hin = agg[:n]

    return _pool_call(agg[:n], b3.reshape(1, d), batch.reshape(n, 1),
                      lin_W, lin_b.reshape(1, -1), NB, ngraphs)
